# baseline (speedup 1.0000x reference)
import functools

import jax
import jax.numpy as jnp
from jax import lax
from jax.experimental import pallas as pl
from jax.experimental.pallas import tpu as pltpu
from jax.experimental.pallas import tpu_sc as plsc

B = 4096
D = 64
P = 4
K = 256
DC = 16
NC = 2
NS = 16
NW = NC * NS
BPW = B // NW

_MESH = plsc.VectorSubcoreMesh(core_axis_name="c", subcore_axis_name="s")


@functools.partial(
    pl.kernel,
    mesh=_MESH,
    compiler_params=pltpu.CompilerParams(use_tc_tiling_on_sc=False, needs_layout_passes=False),
    out_type=[
        jax.ShapeDtypeStruct((B, D), jnp.float32),
        jax.ShapeDtypeStruct((B, D), jnp.float32),
    ],
    scratch_types=[
        pltpu.VMEM((BPW,), jnp.int32),
        pltpu.VMEM((BPW,), jnp.int32),
        pltpu.VMEM((BPW, D), jnp.float32),
        pltpu.VMEM((BPW, D), jnp.float32),
        pltpu.VMEM((BPW, D), jnp.float32),
        pltpu.VMEM((BPW, D), jnp.float32),
        pltpu.SemaphoreType.DMA,
        pltpu.SemaphoreType.DMA,
        pltpu.SemaphoreType.DMA,
    ],
)
def _sc_gather(uid_hbm, iid_hbm, utab_hbm, imu_hbm, ilv_hbm, eps_hbm,
               u_out, iv_out,
               uid_v, iid_v, u_v, mu_v, lv_v, eps_v, sem_u, sem_mu, sem_lv):
    wid = lax.axis_index("s") * NC + lax.axis_index("c")
    base = wid * BPW
    pltpu.sync_copy(uid_hbm.at[pl.ds(base, BPW)], uid_v)
    pltpu.sync_copy(iid_hbm.at[pl.ds(base, BPW)], iid_v)
    cp_u = pltpu.async_copy(utab_hbm.at[uid_v], u_v, sem_u)
    cp_mu = pltpu.async_copy(imu_hbm.at[iid_v], mu_v, sem_mu)
    cp_lv = pltpu.async_copy(ilv_hbm.at[iid_v], lv_v, sem_lv)
    pltpu.sync_copy(eps_hbm.at[pl.ds(base, BPW)], eps_v)
    cp_u.wait()
    pltpu.sync_copy(u_v, u_out.at[pl.ds(base, BPW)])
    cp_mu.wait()
    cp_lv.wait()

    def body(i, carry):
        for c in range(D // 16):
            sl = (i, pl.ds(c * 16, 16))
            mu_v[sl] = mu_v[sl] + eps_v[sl] * jnp.exp(0.5 * lv_v[sl])
        return carry

    lax.fori_loop(0, BPW, body, 0)
    pltpu.sync_copy(mu_v, iv_out.at[pl.ds(base, BPW)])


BB = 1024


def _assign_body(u_ref, c_ref, g_ref, idx_ref):
    u = u_ref[...]
    for p in range(P):
        up = u[:, p * DC:(p + 1) * DC]
        cp = c_ref[p]
        uc = lax.dot_general(up, cp, (((1,), (1,)), ((), ())),
                             precision=lax.Precision.HIGHEST)
        u2 = jnp.sum(up * up, axis=1, keepdims=True)
        c2 = jnp.sum(cp * cp, axis=1)[None, :]
        d2 = jnp.maximum(u2 - 2.0 * uc + c2, 0.0) + 1e-12
        score = g_ref[:, p, :] - jnp.sqrt(d2)
        idx_ref[p, :] = jnp.argmax(score, axis=-1).astype(jnp.int32) + p * K


def _tc_assign(u, centroids, gumbel):
    return pl.pallas_call(
        _assign_body,
        grid=(B // BB,),
        in_specs=[
            pl.BlockSpec((BB, D), lambda i: (i, 0)),
            pl.BlockSpec((P, K, DC), lambda i: (0, 0, 0)),
            pl.BlockSpec((BB, P, K), lambda i: (i, 0, 0)),
        ],
        out_specs=pl.BlockSpec((P, BB), lambda i: (0, i)),
        out_shape=jax.ShapeDtypeStruct((P, B), jnp.int32),
    )(u, centroids, gumbel)


@functools.partial(
    pl.kernel,
    mesh=_MESH,
    compiler_params=pltpu.CompilerParams(use_tc_tiling_on_sc=False, needs_layout_passes=False),
    out_type=jax.ShapeDtypeStruct((B,), jnp.float32),
    scratch_types=[
        pltpu.VMEM((P, BPW), jnp.int32),
        pltpu.VMEM((P, BPW, DC), jnp.float32),
        pltpu.VMEM((BPW, D), jnp.float32),
        pltpu.VMEM((BPW,), jnp.float32),
        pltpu.SemaphoreType.DMA,
    ],
)
def _sc_combine(idx_hbm, ctab_hbm, iv_hbm, out_hbm,
                idx_v, q_v, iv_v, out_v, sem):
    wid = lax.axis_index("s") * NC + lax.axis_index("c")
    base = wid * BPW
    for p in range(P):
        pltpu.sync_copy(idx_hbm.at[p, pl.ds(base, BPW)], idx_v.at[p])
    pltpu.sync_copy(iv_hbm.at[pl.ds(base, BPW)], iv_v)
    cps = [pltpu.async_copy(ctab_hbm.at[idx_v.at[p]], q_v.at[p], sem)
           for p in range(P)]
    for cp in cps:
        cp.wait()

    lane = lax.iota(jnp.int32, 16)

    def body(ci, carry):
        out_chunk = jnp.zeros((16,), jnp.float32)
        for j in range(16):
            i = ci * 16 + j
            acc = q_v[0, i, :] * iv_v[i, pl.ds(0, DC)]
            for p in range(1, P):
                acc = acc + q_v[p, i, :] * iv_v[i, pl.ds(p * DC, DC)]
            out_chunk = jnp.where(lane == j, jnp.sum(acc), out_chunk)
        out_v[pl.ds(pl.multiple_of(ci * 16, 16), 16)] = out_chunk
        return carry

    lax.fori_loop(0, BPW // 16, body, 0)
    pltpu.sync_copy(out_v, out_hbm.at[pl.ds(base, BPW)])


def kernel(user_id, item_id, user_table, centroids, item_mu, item_logvar,
           eps, gumbel):
    u, iv = _sc_gather(user_id, item_id, user_table, item_mu, item_logvar,
                       eps)
    idx = _tc_assign(u, centroids, gumbel)
    ctab = centroids.reshape(P * K, DC)
    return _sc_combine(idx, ctab, iv)


# speedup vs baseline: 2.8823x; 2.8823x over previous
import functools

import jax
import jax.numpy as jnp
from jax import lax
from jax.experimental import pallas as pl
from jax.experimental.pallas import tpu as pltpu
from jax.experimental.pallas import tpu_sc as plsc

B = 4096
D = 64
P = 4
K = 256
DC = 16
NC = 2
NS = 16
NW = NC * NS
BPW = B // NW
NBUF = 8

_MESH = plsc.VectorSubcoreMesh(core_axis_name="c", subcore_axis_name="s")


@functools.partial(
    pl.kernel,
    mesh=_MESH,
    compiler_params=pltpu.CompilerParams(needs_layout_passes=False),
    out_type=jax.ShapeDtypeStruct((D, B), jnp.float32),
    scratch_types=[
        pltpu.VMEM((BPW,), jnp.int32),
        pltpu.VMEM((NBUF, D, 128), jnp.float32),
        pltpu.VMEM((D, BPW), jnp.float32),
    ] + [pltpu.SemaphoreType.DMA] * NBUF,
)
def _sc_user(uid_hbm, utT_hbm, uT_out, uid_v, ring_v, u_c, *sems):
    wid = lax.axis_index("s") * NC + lax.axis_index("c")
    base = wid * BPW
    pltpu.sync_copy(uid_hbm.at[pl.ds(base, BPW)], uid_v)
    lane = lax.iota(jnp.int32, 16)

    def fire(ru, slot):
        start = pl.multiple_of((ru >> 7) * 128, 128)
        for b in range(NBUF):
            @pl.when(slot == b)
            def _():
                pltpu.async_copy(utT_hbm.at[:, pl.ds(start, 128)],
                                 ring_v.at[b], sems[b])

    def extract(ru, slot, col):
        rc = jnp.bitwise_and(ru, 127)
        for b in range(NBUF):
            @pl.when(slot == b)
            def _():
                pltpu.make_async_copy(utT_hbm.at[:, pl.ds(0, 128)],
                                      ring_v.at[b], sems[b]).wait()
                for c4 in range(D // 16):
                    d_idx = c4 * 16 + lane
                    vals = plsc.load_gather(
                        ring_v.at[b], [d_idx, jnp.full((16,), rc, jnp.int32)])
                    plsc.store_scatter(
                        u_c, [d_idx, jnp.full((16,), col, jnp.int32)], vals)

    def sb(s, carry):
        ub = uid_v[pl.ds(pl.multiple_of(s * 16, 16), 16)]
        for j in range(NBUF):
            fire(ub[j], j)
        for j in range(NBUF):
            extract(ub[j], j, s * 16 + j)
            fire(ub[j + NBUF], j)
        for j in range(NBUF):
            extract(ub[j + NBUF], j, s * 16 + j + NBUF)
        return carry

    lax.fori_loop(0, BPW // 16, sb, 0)
    pltpu.sync_copy(u_c, uT_out.at[:, pl.ds(base, BPW)])


@functools.partial(
    pl.kernel,
    mesh=_MESH,
    compiler_params=pltpu.CompilerParams(use_tc_tiling_on_sc=False,
                                         needs_layout_passes=False),
    out_type=jax.ShapeDtypeStruct((B, D), jnp.float32),
    scratch_types=[
        pltpu.VMEM((BPW,), jnp.int32),
        pltpu.VMEM((BPW, D), jnp.float32),
        pltpu.VMEM((BPW, D), jnp.float32),
        pltpu.VMEM((BPW, D), jnp.float32),
        pltpu.SemaphoreType.DMA,
        pltpu.SemaphoreType.DMA,
    ],
)
def _sc_item(iid_hbm, imu_hbm, ilv_hbm, eps_hbm, iv_out,
             iid_v, mu_v, lv_v, eps_v, sem_mu, sem_lv):
    wid = lax.axis_index("s") * NC + lax.axis_index("c")
    base = wid * BPW
    pltpu.sync_copy(iid_hbm.at[pl.ds(base, BPW)], iid_v)
    cp_mu = pltpu.async_copy(imu_hbm.at[iid_v], mu_v, sem_mu)
    cp_lv = pltpu.async_copy(ilv_hbm.at[iid_v], lv_v, sem_lv)
    pltpu.sync_copy(eps_hbm.at[pl.ds(base, BPW)], eps_v)
    cp_mu.wait()
    cp_lv.wait()

    def body(i, carry):
        for c in range(D // 16):
            sl = (i, pl.ds(c * 16, 16))
            mu_v[sl] = mu_v[sl] + eps_v[sl] * jnp.exp(0.5 * lv_v[sl])
        return carry

    lax.fori_loop(0, BPW, body, 0)
    pltpu.sync_copy(mu_v, iv_out.at[pl.ds(base, BPW)])


BB = 1024


def _dense_body(uT_ref, c_ref, g_ref, iv_ref, out_ref):
    lane_k = lax.broadcasted_iota(jnp.int32, (BB, K), 1)
    acc = None
    for p in range(P):
        uTp = uT_ref[p * DC:(p + 1) * DC, :]
        cp = c_ref[p]
        uc = lax.dot_general(uTp, cp, (((0,), (1,)), ((), ())),
                             precision=lax.Precision.HIGHEST)
        u2 = jnp.sum(uTp * uTp, axis=0)[:, None]
        c2 = jnp.sum(cp * cp, axis=1)[None, :]
        d2 = jnp.maximum(u2 - 2.0 * uc + c2, 0.0) + 1e-12
        score = g_ref[:, p, :] - jnp.sqrt(d2)
        am = jnp.argmax(score, axis=-1).astype(jnp.int32)
        oh = (lane_k == am[:, None]).astype(jnp.float32)
        qp = lax.dot_general(oh, cp, (((1,), (0,)), ((), ())),
                             precision=lax.Precision.HIGHEST)
        part = jnp.sum(qp * iv_ref[:, p * DC:(p + 1) * DC], axis=1)
        acc = part if acc is None else acc + part
    out_ref[...] = acc


def _tc_dense(uT, centroids, gumbel, iv):
    return pl.pallas_call(
        _dense_body,
        grid=(B // BB,),
        in_specs=[
            pl.BlockSpec((D, BB), lambda i: (0, i)),
            pl.BlockSpec((P, K, DC), lambda i: (0, 0, 0)),
            pl.BlockSpec((BB, P, K), lambda i: (i, 0, 0)),
            pl.BlockSpec((BB, D), lambda i: (i, 0)),
        ],
        out_specs=pl.BlockSpec((BB,), lambda i: (i,)),
        out_shape=jax.ShapeDtypeStruct((B,), jnp.float32),
    )(uT, centroids, gumbel, iv)


def kernel(user_id, item_id, user_table, centroids, item_mu, item_logvar,
           eps, gumbel):
    uT = _sc_user(user_id, user_table.T)
    iv = _sc_item(item_id, item_mu, item_logvar, eps)
    return _tc_dense(uT, centroids, gumbel, iv)


# speedup vs baseline: 3.8005x; 1.3186x over previous
import functools

import jax
import jax.numpy as jnp
from jax import lax
from jax.experimental import pallas as pl
from jax.experimental.pallas import tpu as pltpu
from jax.experimental.pallas import tpu_sc as plsc

B = 4096
D = 64
P = 4
K = 256
DC = 16
NI = 100000
NC = 2
NS = 16
NW = NC * NS
BPW = B // NW
NBUF = 8

_MESH = plsc.VectorSubcoreMesh(core_axis_name="c", subcore_axis_name="s")


NB_T = 8192


def _prep_body(muT_ref, lvT_ref, out_ref):
    out_ref[:, 0:D] = muT_ref[...].T
    out_ref[:, D:2 * D] = lvT_ref[...].T


def _tc_prep(muT, lvT):
    return pl.pallas_call(
        _prep_body,
        grid=((NI + NB_T - 1) // NB_T,),
        in_specs=[
            pl.BlockSpec((D, NB_T), lambda i: (0, i)),
            pl.BlockSpec((D, NB_T), lambda i: (0, i)),
        ],
        out_specs=pl.BlockSpec((NB_T, 2 * D), lambda i: (i, 0)),
        out_shape=jax.ShapeDtypeStruct((NI, 2 * D), jnp.float32),
    )(muT, lvT)


@functools.partial(
    pl.kernel,
    mesh=_MESH,
    compiler_params=pltpu.CompilerParams(needs_layout_passes=False),
    out_type=[
        jax.ShapeDtypeStruct((D, B), jnp.float32),
        jax.ShapeDtypeStruct((B, 2 * D), jnp.float32),
    ],
    scratch_types=[
        pltpu.VMEM((BPW,), jnp.int32),
        pltpu.VMEM((BPW,), jnp.int32),
        pltpu.VMEM((NBUF, D, 128), jnp.float32),
        pltpu.VMEM((D, BPW), jnp.float32),
        pltpu.VMEM((BPW, 2 * D), jnp.float32),
        pltpu.SemaphoreType.DMA,
    ] + [pltpu.SemaphoreType.DMA] * NBUF,
)
def _sc_gather(uid_hbm, iid_hbm, utT_hbm, mulv_hbm, uT_out, ms_out,
               uid_v, iid_v, ring_v, u_c, ms_v, sem_g, *sems):
    wid = lax.axis_index("s") * NC + lax.axis_index("c")
    base = wid * BPW
    pltpu.sync_copy(uid_hbm.at[pl.ds(base, BPW)], uid_v)
    pltpu.sync_copy(iid_hbm.at[pl.ds(base, BPW)], iid_v)
    cp_items = pltpu.async_copy(mulv_hbm.at[iid_v], ms_v, sem_g)
    lane = lax.iota(jnp.int32, 16)

    def fire(ru, slot):
        start = pl.multiple_of((ru >> 7) * 128, 128)
        for b in range(NBUF):
            @pl.when(slot == b)
            def _():
                pltpu.async_copy(utT_hbm.at[:, pl.ds(start, 128)],
                                 ring_v.at[b], sems[b])

    def extract(ru, slot, col):
        rc = jnp.bitwise_and(ru, 127)
        for b in range(NBUF):
            @pl.when(slot == b)
            def _():
                pltpu.make_async_copy(utT_hbm.at[:, pl.ds(0, 128)],
                                      ring_v.at[b], sems[b]).wait()
                for c4 in range(D // 16):
                    d_idx = c4 * 16 + lane
                    vals = plsc.load_gather(
                        ring_v.at[b], [d_idx, jnp.full((16,), rc, jnp.int32)])
                    plsc.store_scatter(
                        u_c, [d_idx, jnp.full((16,), col, jnp.int32)], vals)

    def sb(s, carry):
        ub = uid_v[pl.ds(pl.multiple_of(s * 16, 16), 16)]
        for j in range(NBUF):
            fire(ub[j], j)
        for j in range(NBUF):
            extract(ub[j], j, s * 16 + j)
            fire(ub[j + NBUF], j)
        for j in range(NBUF):
            extract(ub[j + NBUF], j, s * 16 + j + NBUF)
        return carry

    lax.fori_loop(0, BPW // 16, sb, 0)
    pltpu.sync_copy(u_c, uT_out.at[:, pl.ds(base, BPW)])
    cp_items.wait()
    pltpu.sync_copy(ms_v, ms_out.at[pl.ds(base, BPW)])


BB = 1024


def _dense_body(uT_ref, c_ref, g_ref, ms_ref, eps_ref, out_ref):
    lane_k = lax.broadcasted_iota(jnp.int32, (BB, K), 1)
    acc = None
    for p in range(P):
        uTp = uT_ref[p * DC:(p + 1) * DC, :]
        cp = c_ref[p]
        uc = lax.dot_general(uTp, cp, (((0,), (1,)), ((), ())),
                             precision=lax.Precision.HIGHEST)
        u2 = jnp.sum(uTp * uTp, axis=0)[:, None]
        c2 = jnp.sum(cp * cp, axis=1)[None, :]
        d2 = jnp.maximum(u2 - 2.0 * uc + c2, 0.0) + 1e-12
        score = g_ref[:, p, :] - jnp.sqrt(d2)
        am = jnp.argmax(score, axis=-1).astype(jnp.int32)
        oh = (lane_k == am[:, None]).astype(jnp.float32)
        qp = lax.dot_general(oh, cp, (((1,), (0,)), ((), ())),
                             precision=lax.Precision.HIGHEST)
        sl = slice(p * DC, (p + 1) * DC)
        iv_p = (ms_ref[:, p * DC:(p + 1) * DC]
                + eps_ref[:, sl] * jnp.exp(0.5 * ms_ref[:, D + p * DC:
                                                        D + (p + 1) * DC]))
        part = jnp.sum(qp * iv_p, axis=1)
        acc = part if acc is None else acc + part
    out_ref[...] = acc


def _tc_dense(uT, centroids, gumbel, ms, eps):
    return pl.pallas_call(
        _dense_body,
        grid=(B // BB,),
        in_specs=[
            pl.BlockSpec((D, BB), lambda i: (0, i)),
            pl.BlockSpec((P, K, DC), lambda i: (0, 0, 0)),
            pl.BlockSpec((BB, P, K), lambda i: (i, 0, 0)),
            pl.BlockSpec((BB, 2 * D), lambda i: (i, 0)),
            pl.BlockSpec((BB, D), lambda i: (i, 0)),
        ],
        out_specs=pl.BlockSpec((BB,), lambda i: (i,)),
        out_shape=jax.ShapeDtypeStruct((B,), jnp.float32),
    )(uT, centroids, gumbel, ms, eps)


def kernel(user_id, item_id, user_table, centroids, item_mu, item_logvar,
           eps, gumbel):
    mulv = _tc_prep(item_mu.T, item_logvar.T)
    uT, ms = _sc_gather(user_id, item_id, user_table.T, mulv)
    return _tc_dense(uT, centroids, gumbel, ms, eps)
